# Initial kernel scaffold; baseline (speedup 1.0000x reference)
import jax
import jax.numpy as jnp
from jax.experimental import pallas as pl


def kernel(input_tensor, center_positions, mask, crop_mask):
    raise NotImplementedError("write your pallas kernel here")


# speedup vs baseline: 2.9333x; 2.9333x over previous
import jax
import jax.numpy as jnp
from jax import lax
from jax.experimental import pallas as pl
from jax.experimental.pallas import tpu as pltpu
from jax.experimental.pallas import tpu_sc as plsc

BATCH = 128
CHANNELS = 32
ENV = 128
CS = 33
CC = 16
NC = 2
NS = 16
NW = NC * NS
BPW = BATCH // NW
CH_CHUNK = 8
N_CHUNKS = CHANNELS // CH_CHUNK
BUF_W = 176
MROW_W = 48
CH_OUT = CS * CS
BLK_OUT = CH_CHUNK * CH_OUT
OBUF_W = BLK_OUT + 24


def _sc_body(x_hbm, scal_hbm, mpad_hbm, out_hbm, buf, obuf, scal_v, mask_v):
    wid = lax.axis_index("s") * NC + lax.axis_index("c")
    lane = lax.iota(jnp.int32, 16)

    pltpu.sync_copy(mpad_hbm, mask_v)
    pltpu.sync_copy(scal_hbm.at[pl.ds(wid * BPW, BPW)], scal_v)

    def zero_row(i, _):
        for ch in range(CH_CHUNK):
            for k in range(BUF_W // 16):
                buf[ch, i, pl.ds(16 * k, 16)] = jnp.zeros((16,), jnp.float32)
        return _

    lax.fori_loop(0, CS, zero_row, None)

    def batch_body(bi, _):
        srow = scal_v[bi]
        rs2 = srow[0]
        du = srow[1]
        vv = srow[2]
        b = wid * BPW + bi

        def chunk_body(cc, _):
            pltpu.sync_copy(
                x_hbm.at[b, pl.ds(cc * CH_CHUNK, CH_CHUNK), pl.ds(rs2, CS), :],
                buf.at[:, :, pl.ds(CC, ENV)],
            )

            def ch_body(ch, _):
                def compute_row(i, _):
                    jb = jnp.clip(i + du, 0, CS - 1)
                    img_row = rs2 + du + i
                    vi = jnp.where(
                        jnp.logical_and(img_row >= 0, img_row < ENV), 1.0, 0.0
                    ).astype(jnp.float32)
                    obase = ch * CH_OUT + i * CS
                    for k in range(3):
                        mk = mask_v[i, pl.ds(16 * k, 16)] * vi
                        obuf[pl.ds(obase + 16 * k, 16)] = (
                            buf[ch, jb, pl.ds(vv + 16 * k, 16)] * mk
                        )
                    return _

                lax.fori_loop(0, CS, compute_row, None)
                return _

            lax.fori_loop(0, CH_CHUNK, ch_body, None)

            pltpu.sync_copy(
                obuf.at[pl.ds(0, BLK_OUT)],
                out_hbm.at[b, pl.ds(cc * BLK_OUT, BLK_OUT)],
            )
            return _

        lax.fori_loop(0, N_CHUNKS, chunk_body, None)
        return _

    lax.fori_loop(0, BPW, batch_body, None)


@jax.jit
def _hexcrop_sc(input_tensor, scal, mpad):
    mesh = plsc.VectorSubcoreMesh(
        core_axis_name="c", subcore_axis_name="s", num_cores=NC, num_subcores=NS
    )
    f = pl.kernel(
        _sc_body,
        out_type=jax.ShapeDtypeStruct((BATCH, CHANNELS * CH_OUT), jnp.float32),
        mesh=mesh,
        scratch_types=[
            pltpu.VMEM((CH_CHUNK, CS, BUF_W), jnp.float32),
            pltpu.VMEM((OBUF_W,), jnp.float32),
            pltpu.VMEM((BPW, 16), jnp.int32),
            pltpu.VMEM((CS, MROW_W), jnp.float32),
        ],
        compiler_params=pltpu.CompilerParams(use_tc_tiling_on_sc=False),
    )
    return f(input_tensor, scal, mpad)


def kernel(input_tensor, center_positions, mask, crop_mask):
    u = center_positions[:, 0].astype(jnp.int32)
    v = center_positions[:, 1].astype(jnp.int32)
    rs2 = jnp.clip(u - CC, 0, ENV - CS)
    du = (u - CC) - rs2
    scal = jnp.zeros((BATCH, 16), jnp.int32)
    scal = scal.at[:, 0].set(rs2).at[:, 1].set(du).at[:, 2].set(v)

    m_eff = jnp.where(mask, crop_mask, jnp.ones_like(crop_mask))
    mpad = jnp.zeros((CS, MROW_W), jnp.float32).at[:, :CS].set(m_eff)

    out = _hexcrop_sc(input_tensor, scal, mpad)
    out = out.reshape(BATCH, CHANNELS, CS, CS)
    return (out, crop_mask)


# speedup vs baseline: 3.0482x; 1.0392x over previous
import jax
import jax.numpy as jnp
from jax import lax
from jax.experimental import pallas as pl
from jax.experimental.pallas import tpu as pltpu
from jax.experimental.pallas import tpu_sc as plsc

BATCH = 128
CHANNELS = 32
ENV = 128
CS = 33
CC = 16
NC = 2
NS = 16
NW = NC * NS
BPW = BATCH // NW
CH_CHUNK = 16
N_CHUNKS = CHANNELS // CH_CHUNK
FETCH_W = 48
BUF_W = 96
MROW_W = 48
CH_OUT = CS * CS
BLK_OUT = CH_CHUNK * CH_OUT
OBUF_W = BLK_OUT + 24


def _sc_body(x_hbm, scal_hbm, mpad_hbm, out_hbm, buf, obuf, scal_v, mask_v):
    wid = lax.axis_index("s") * NC + lax.axis_index("c")

    pltpu.sync_copy(mpad_hbm, mask_v)
    pltpu.sync_copy(scal_hbm.at[pl.ds(wid * BPW, BPW)], scal_v)

    def zero_row(i, _):
        for ch in range(CH_CHUNK):
            for k in (0, 4, 5):
                buf[ch, i, pl.ds(16 * k, 16)] = jnp.zeros((16,), jnp.float32)
        return _

    lax.fori_loop(0, CS, zero_row, None)

    def batch_body(bi, _):
        srow = scal_v[bi]
        rs2 = srow[0]
        du = srow[1]
        cs2 = pl.multiple_of(srow[2], 16)
        dv = srow[3]
        b = wid * BPW + bi

        def chunk_body(cc, _):
            pltpu.sync_copy(
                x_hbm.at[
                    b,
                    pl.ds(cc * CH_CHUNK, CH_CHUNK),
                    pl.ds(rs2, CS),
                    pl.ds(cs2, FETCH_W),
                ],
                buf.at[:, :, pl.ds(CC, FETCH_W)],
            )

            def ch_body(ch, _):
                def compute_row(i, _):
                    jb = jnp.clip(i + du, 0, CS - 1)
                    img_row = rs2 + du + i
                    vi = jnp.where(
                        jnp.logical_and(img_row >= 0, img_row < ENV), 1.0, 0.0
                    ).astype(jnp.float32)
                    obase = ch * CH_OUT + i * CS
                    rbase = CC + dv
                    for k in range(3):
                        mk = mask_v[i, pl.ds(16 * k, 16)] * vi
                        obuf[pl.ds(obase + 16 * k, 16)] = (
                            buf[ch, jb, pl.ds(rbase + 16 * k, 16)] * mk
                        )
                    return _

                lax.fori_loop(0, CS, compute_row, None)
                return _

            lax.fori_loop(0, CH_CHUNK, ch_body, None)

            pltpu.sync_copy(
                obuf.at[pl.ds(0, BLK_OUT)],
                out_hbm.at[b, pl.ds(cc * BLK_OUT, BLK_OUT)],
            )
            return _

        lax.fori_loop(0, N_CHUNKS, chunk_body, None)
        return _

    lax.fori_loop(0, BPW, batch_body, None)


@jax.jit
def _hexcrop_sc(input_tensor, scal, mpad):
    mesh = plsc.VectorSubcoreMesh(
        core_axis_name="c", subcore_axis_name="s", num_cores=NC, num_subcores=NS
    )
    f = pl.kernel(
        _sc_body,
        out_type=jax.ShapeDtypeStruct((BATCH, CHANNELS * CH_OUT), jnp.float32),
        mesh=mesh,
        scratch_types=[
            pltpu.VMEM((CH_CHUNK, CS, BUF_W), jnp.float32),
            pltpu.VMEM((OBUF_W,), jnp.float32),
            pltpu.VMEM((BPW, 16), jnp.int32),
            pltpu.VMEM((CS, MROW_W), jnp.float32),
        ],
        compiler_params=pltpu.CompilerParams(use_tc_tiling_on_sc=False),
    )
    return f(input_tensor, scal, mpad)


def kernel(input_tensor, center_positions, mask, crop_mask):
    u = center_positions[:, 0].astype(jnp.int32)
    v = center_positions[:, 1].astype(jnp.int32)
    rs2 = jnp.clip(u - CC, 0, ENV - CS)
    du = (u - CC) - rs2
    cs2 = jnp.clip(v - CC, 0, ENV - FETCH_W) & ~15
    dv = (v - CC) - cs2
    scal = jnp.zeros((BATCH, 16), jnp.int32)
    scal = (
        scal.at[:, 0].set(rs2).at[:, 1].set(du).at[:, 2].set(cs2).at[:, 3].set(dv)
    )

    m_eff = jnp.where(mask, crop_mask, jnp.ones_like(crop_mask))
    mpad = jnp.zeros((CS, MROW_W), jnp.float32).at[:, :CS].set(m_eff)

    out = _hexcrop_sc(input_tensor, scal, mpad)
    out = out.reshape(BATCH, CHANNELS, CS, CS)
    return (out, crop_mask)


# speedup vs baseline: 3.7998x; 1.2466x over previous
import numpy as np

import jax
import jax.numpy as jnp
from jax import lax
from jax.experimental import pallas as pl
from jax.experimental.pallas import tpu as pltpu
from jax.experimental.pallas import tpu_sc as plsc

BATCH = 128
CHANNELS = 32
ENV = 128
CS = 33
CC = 16
NC = 2
NS = 16
NW = NC * NS
BPW = BATCH // NW
CH_CHUNK = 16
N_CHUNKS = CHANNELS // CH_CHUNK
FETCH_W = 48
BUF_W = 96
CH_OUT = CS * CS
BLK_OUT = CH_CHUNK * CH_OUT
NT = (CH_OUT + 15) // 16
OBUF_W = BLK_OUT + 16

_P = np.arange(NT * 16)
_Q = _P % CH_OUT
_I = (_Q // CS).astype(np.int32)
_J = (_Q % CS).astype(np.int32)
_COMBO = jnp.asarray((_I | (_J << 8)).reshape(NT, 16))


def _sc_body(
    x_hbm, scal_hbm, combo_hbm, mtab_hbm, out_hbm,
    buf, obuf, scal_v, combo_v, mtab_v, jctab, vmtab,
):
    wid = lax.axis_index("s") * NC + lax.axis_index("c")

    pltpu.sync_copy(combo_hbm, combo_v)
    pltpu.sync_copy(mtab_hbm, mtab_v)
    pltpu.sync_copy(scal_hbm.at[pl.ds(wid * BPW, BPW)], scal_v)

    lane = lax.iota(jnp.int32, 16)
    chd = jnp.where(lane > 0, 1, 0)

    def zero_row(i, _):
        for ch in range(CH_CHUNK):
            for k in (0, 4, 5):
                buf[ch, i, pl.ds(16 * k, 16)] = jnp.zeros((16,), jnp.float32)
        return _

    lax.fori_loop(0, CS, zero_row, None)

    def batch_body(bi, _):
        srow = scal_v[bi]
        rs2 = srow[0]
        du = srow[1]
        cs2 = pl.multiple_of(srow[2], 16)
        dv = srow[3]
        b = wid * BPW + bi
        rsdu = rs2 + du

        def table_body(t, _):
            c = combo_v[t]
            i_vec = c & 255
            j_vec = c >> 8
            jb = jnp.clip(i_vec + du, 0, CS - 1)
            col = j_vec + (CC + dv)
            jctab[t] = jb | (col << 8)
            img = i_vec + rsdu
            valid = jnp.logical_and(img >= 0, img < ENV)
            vmtab[t] = jnp.where(valid, mtab_v[t], 0.0)
            return _

        lax.fori_loop(0, NT, table_body, None)

        def chunk_body(cc, _):
            pltpu.sync_copy(
                x_hbm.at[
                    b,
                    pl.ds(cc * CH_CHUNK, CH_CHUNK),
                    pl.ds(rs2, CS),
                    pl.ds(cs2, FETCH_W),
                ],
                buf.at[:, :, pl.ds(CC, FETCH_W)],
            )

            def ch_body(ch, _):
                chsplat = jnp.full((16,), 0, jnp.int32) + ch
                chbase = ch * CH_OUT

                @plsc.parallel_loop(0, NT - 1, unroll=4)
                def chunk_loop(t):
                    pk = jctab[t]
                    jb = pk & 255
                    col = pk >> 8
                    g = plsc.load_gather(buf, [chsplat, jb, col])
                    obuf[pl.ds(chbase + 16 * t, 16)] = g * vmtab[t]

                pk = jctab[NT - 1]
                jb = pk & 255
                col = pk >> 8
                chv = jnp.minimum(chsplat + chd, CH_CHUNK - 1)
                g = plsc.load_gather(buf, [chv, jb, col])
                obuf[pl.ds(chbase + 16 * (NT - 1), 16)] = g * vmtab[NT - 1]
                return _

            lax.fori_loop(0, CH_CHUNK, ch_body, None)

            pltpu.sync_copy(
                obuf.at[pl.ds(0, BLK_OUT)],
                out_hbm.at[b, pl.ds(cc * BLK_OUT, BLK_OUT)],
            )
            return _

        lax.fori_loop(0, N_CHUNKS, chunk_body, None)
        return _

    lax.fori_loop(0, BPW, batch_body, None)


@jax.jit
def _hexcrop_sc(input_tensor, scal, mtab):
    mesh = plsc.VectorSubcoreMesh(
        core_axis_name="c", subcore_axis_name="s", num_cores=NC, num_subcores=NS
    )
    f = pl.kernel(
        _sc_body,
        out_type=jax.ShapeDtypeStruct((BATCH, CHANNELS * CH_OUT), jnp.float32),
        mesh=mesh,
        scratch_types=[
            pltpu.VMEM((CH_CHUNK, CS, BUF_W), jnp.float32),
            pltpu.VMEM((OBUF_W,), jnp.float32),
            pltpu.VMEM((BPW, 16), jnp.int32),
            pltpu.VMEM((NT, 16), jnp.int32),
            pltpu.VMEM((NT, 16), jnp.float32),
            pltpu.VMEM((NT, 16), jnp.int32),
            pltpu.VMEM((NT, 16), jnp.float32),
        ],
        compiler_params=pltpu.CompilerParams(
            use_tc_tiling_on_sc=False, needs_layout_passes=False
        ),
    )
    return f(input_tensor, scal, _COMBO, mtab)


def kernel(input_tensor, center_positions, mask, crop_mask):
    u = center_positions[:, 0].astype(jnp.int32)
    v = center_positions[:, 1].astype(jnp.int32)
    rs2 = jnp.clip(u - CC, 0, ENV - CS)
    du = (u - CC) - rs2
    cs2 = jnp.clip(v - CC, 0, ENV - FETCH_W) & ~15
    dv = (v - CC) - cs2
    scal = jnp.zeros((BATCH, 16), jnp.int32)
    scal = (
        scal.at[:, 0].set(rs2).at[:, 1].set(du).at[:, 2].set(cs2).at[:, 3].set(dv)
    )

    m_eff = jnp.where(mask, crop_mask, jnp.ones_like(crop_mask))
    mtab = m_eff[jnp.asarray(_I), jnp.asarray(_J)].reshape(NT, 16)

    out = _hexcrop_sc(input_tensor, scal, mtab)
    out = out.reshape(BATCH, CHANNELS, CS, CS)
    return (out, crop_mask)


# speedup vs baseline: 3.9420x; 1.0374x over previous
import numpy as np

import jax
import jax.numpy as jnp
from jax import lax
from jax.experimental import pallas as pl
from jax.experimental.pallas import tpu as pltpu
from jax.experimental.pallas import tpu_sc as plsc

BATCH = 128
CHANNELS = 32
ENV = 128
CS = 33
CC = 16
NC = 2
NS = 16
NW = NC * NS
BPW = BATCH // NW
CH_CHUNK = 16
N_CHUNKS = CHANNELS // CH_CHUNK
FETCH_W = 48
BUF_W = 96
CH_OUT = CS * CS
BLK_OUT = CH_CHUNK * CH_OUT
NT = (CH_OUT + 15) // 16
OBUF_W = BLK_OUT + 16

_P = np.arange(NT * 16)
_Q = _P % CH_OUT
_I = (_Q // CS).astype(np.int32)
_J = (_Q % CS).astype(np.int32)
_COMBO = jnp.asarray((_I | (_J << 8)).reshape(NT, 16))


def _sc_body(
    x_hbm, scal_hbm, combo_hbm, mtab_hbm, out_hbm,
    buf, obuf, scal_v, combo_v, mtab_v, jctab, vmtab,
):
    wid = lax.axis_index("s") * NC + lax.axis_index("c")

    pltpu.sync_copy(combo_hbm, combo_v)
    pltpu.sync_copy(mtab_hbm, mtab_v)
    pltpu.sync_copy(scal_hbm.at[pl.ds(wid * BPW, BPW)], scal_v)

    lane = lax.iota(jnp.int32, 16)
    chd = jnp.where(lane > 0, 1, 0)

    def zero_row(i, _):
        for ch in range(CH_CHUNK):
            for k in (0, 4, 5):
                buf[ch, i, pl.ds(16 * k, 16)] = jnp.zeros((16,), jnp.float32)
        return _

    lax.fori_loop(0, CS, zero_row, None)

    def batch_body(bi, _):
        srow = scal_v[bi]
        rs2 = srow[0]
        du = srow[1]
        cs2 = pl.multiple_of(srow[2], 16)
        dv = srow[3]
        b = wid * BPW + bi
        rsdu = rs2 + du

        def table_body(t, _):
            c = combo_v[t]
            i_vec = c & 255
            j_vec = c >> 8
            jb = jnp.clip(i_vec + du, 0, CS - 1)
            col = j_vec + (CC + dv)
            jctab[t] = jb | (col << 8)
            img = i_vec + rsdu
            valid = jnp.logical_and(img >= 0, img < ENV)
            vmtab[t] = jnp.where(valid, mtab_v[t], 0.0)
            return _

        lax.fori_loop(0, NT, table_body, None)

        def chunk_body(cc, _):
            pltpu.sync_copy(
                x_hbm.at[
                    b,
                    pl.ds(cc * CH_CHUNK, CH_CHUNK),
                    pl.ds(rs2, CS),
                    pl.ds(cs2, FETCH_W),
                ],
                buf.at[:, :, pl.ds(CC, FETCH_W)],
            )

            @plsc.parallel_loop(0, (NT - 1) * CH_CHUNK, unroll=8)
            def chunk_loop(m):
                ch = m & (CH_CHUNK - 1)
                t = m >> 4
                pk = jctab[t]
                jb = pk & 255
                col = pk >> 8
                chsplat = jnp.full((16,), 0, jnp.int32) + ch
                g = plsc.load_gather(buf, [chsplat, jb, col])
                obuf[pl.ds(ch * CH_OUT + 16 * t, 16)] = g * vmtab[t]

            @plsc.parallel_loop(0, CH_CHUNK, unroll=2)
            def tail_loop(ch):
                pk = jctab[NT - 1]
                jb = pk & 255
                col = pk >> 8
                chsplat = jnp.full((16,), 0, jnp.int32) + ch
                chv = jnp.minimum(chsplat + chd, CH_CHUNK - 1)
                g = plsc.load_gather(buf, [chv, jb, col])
                obuf[pl.ds(ch * CH_OUT + 16 * (NT - 1), 16)] = g * vmtab[NT - 1]

            pltpu.sync_copy(
                obuf.at[pl.ds(0, BLK_OUT)],
                out_hbm.at[b, pl.ds(cc * BLK_OUT, BLK_OUT)],
            )
            return _

        lax.fori_loop(0, N_CHUNKS, chunk_body, None)
        return _

    lax.fori_loop(0, BPW, batch_body, None)


@jax.jit
def _hexcrop_sc(input_tensor, scal, mtab):
    mesh = plsc.VectorSubcoreMesh(
        core_axis_name="c", subcore_axis_name="s", num_cores=NC, num_subcores=NS
    )
    f = pl.kernel(
        _sc_body,
        out_type=jax.ShapeDtypeStruct((BATCH, CHANNELS * CH_OUT), jnp.float32),
        mesh=mesh,
        scratch_types=[
            pltpu.VMEM((CH_CHUNK, CS, BUF_W), jnp.float32),
            pltpu.VMEM((OBUF_W,), jnp.float32),
            pltpu.VMEM((BPW, 16), jnp.int32),
            pltpu.VMEM((NT, 16), jnp.int32),
            pltpu.VMEM((NT, 16), jnp.float32),
            pltpu.VMEM((NT, 16), jnp.int32),
            pltpu.VMEM((NT, 16), jnp.float32),
        ],
        compiler_params=pltpu.CompilerParams(
            use_tc_tiling_on_sc=False, needs_layout_passes=False
        ),
    )
    return f(input_tensor, scal, _COMBO, mtab)


def kernel(input_tensor, center_positions, mask, crop_mask):
    u = center_positions[:, 0].astype(jnp.int32)
    v = center_positions[:, 1].astype(jnp.int32)
    rs2 = jnp.clip(u - CC, 0, ENV - CS)
    du = (u - CC) - rs2
    cs2 = jnp.clip(v - CC, 0, ENV - FETCH_W) & ~15
    dv = (v - CC) - cs2
    scal = jnp.zeros((BATCH, 16), jnp.int32)
    scal = (
        scal.at[:, 0].set(rs2).at[:, 1].set(du).at[:, 2].set(cs2).at[:, 3].set(dv)
    )

    m_eff = jnp.where(mask, crop_mask, jnp.ones_like(crop_mask))
    mtab = m_eff[jnp.asarray(_I), jnp.asarray(_J)].reshape(NT, 16)

    out = _hexcrop_sc(input_tensor, scal, mtab)
    out = out.reshape(BATCH, CHANNELS, CS, CS)
    return (out, crop_mask)


# speedup vs baseline: 4.0760x; 1.0340x over previous
import numpy as np

import jax
import jax.numpy as jnp
from jax import lax
from jax.experimental import pallas as pl
from jax.experimental.pallas import tpu as pltpu
from jax.experimental.pallas import tpu_sc as plsc

BATCH = 128
CHANNELS = 32
ENV = 128
CS = 33
CC = 16
NC = 2
NS = 16
NW = NC * NS
BPW = BATCH // NW
CH_CHUNK = 8
N_CHUNKS = CHANNELS // CH_CHUNK
NUNITS = BPW * N_CHUNKS
FETCH_W = 48
BUF_W = 96
CH_OUT = CS * CS
BLK_OUT = CH_CHUNK * CH_OUT
NT = (CH_OUT + 15) // 16
OBUF_W = BLK_OUT + 24

_P = np.arange(NT * 16)
_Q = _P % CH_OUT
_I = (_Q // CS).astype(np.int32)
_J = (_Q % CS).astype(np.int32)
_COMBO = jnp.asarray((_I | (_J << 8)).reshape(NT, 16))


def _sc_body(
    x_hbm, scal_hbm, combo_hbm, mtab_hbm, out_hbm,
    buf, obuf, scal_v, combo_v, mtab_v, jctab, vmtab, sem,
):
    wid = lax.axis_index("s") * NC + lax.axis_index("c")

    pltpu.sync_copy(combo_hbm, combo_v)
    pltpu.sync_copy(mtab_hbm, mtab_v)
    pltpu.sync_copy(scal_hbm.at[pl.ds(wid * BPW, BPW)], scal_v)

    lane = lax.iota(jnp.int32, 16)
    chd = jnp.where(lane > 0, 1, 0)

    def zero_row(i, _):
        for ch in range(2 * CH_CHUNK):
            for k in (0, 4, 5):
                buf[ch, i, pl.ds(16 * k, 16)] = jnp.zeros((16,), jnp.float32)
        return _

    lax.fori_loop(0, CS, zero_row, None)

    def tbl_batch(bi, _):
        srow = scal_v[bi]
        rs2 = srow[0]
        du = srow[1]
        dv = srow[3]
        rsdu = rs2 + du

        def table_body(t, _):
            c = combo_v[t]
            i_vec = c & 255
            j_vec = c >> 8
            jb = jnp.clip(i_vec + du, 0, CS - 1)
            col = j_vec + (CC + dv)
            jctab[bi * NT + t] = jb | (col << 8)
            img = i_vec + rsdu
            valid = jnp.logical_and(img >= 0, img < ENV)
            vmtab[bi * NT + t] = jnp.where(valid, mtab_v[t], 0.0)
            return _

        lax.fori_loop(0, NT, table_body, None)
        return _

    lax.fori_loop(0, BPW, tbl_batch, None)

    def band_copy(u, par):
        bi = u >> 2
        cc = u & (N_CHUNKS - 1)
        srow = scal_v[bi]
        rs2 = srow[0]
        cs2 = pl.multiple_of(srow[2], 16)
        b = wid * BPW + bi
        return pltpu.make_async_copy(
            x_hbm.at[
                b,
                pl.ds(cc * CH_CHUNK, CH_CHUNK),
                pl.ds(rs2, CS),
                pl.ds(cs2, FETCH_W),
            ],
            buf.at[pl.ds(par * CH_CHUNK, CH_CHUNK), :, pl.ds(CC, FETCH_W)],
            sem.at[par],
        )

    band_copy(0, 0).start()

    def unit_body(u, _):
        par = u & 1
        bi = u >> 2
        cc = u & (N_CHUNKS - 1)
        b = wid * BPW + bi
        tb = bi * NT

        @pl.when(u + 1 < NUNITS)
        def _start_next():
            band_copy(u + 1, 1 - par).start()

        band_copy(u, par).wait()

        @plsc.parallel_loop(0, (NT - 1) * CH_CHUNK, unroll=8)
        def chunk_loop(m):
            ch = m & (CH_CHUNK - 1)
            t = m >> 3
            pk = jctab[tb + t]
            jb = pk & 255
            col = pk >> 8
            chsplat = jnp.full((16,), 0, jnp.int32) + (par * CH_CHUNK + ch)
            g = plsc.load_gather(buf, [chsplat, jb, col])
            obuf[pl.ds(ch * CH_OUT + 16 * t, 16)] = g * vmtab[tb + t]

        @plsc.parallel_loop(0, CH_CHUNK, unroll=2)
        def tail_loop(ch):
            pk = jctab[tb + NT - 1]
            jb = pk & 255
            col = pk >> 8
            chsplat = jnp.full((16,), 0, jnp.int32) + (par * CH_CHUNK + ch)
            chv = jnp.minimum(chsplat + chd, par * CH_CHUNK + CH_CHUNK - 1)
            g = plsc.load_gather(buf, [chv, jb, col])
            obuf[pl.ds(ch * CH_OUT + 16 * (NT - 1), 16)] = g * vmtab[tb + NT - 1]

        pltpu.sync_copy(
            obuf.at[pl.ds(0, BLK_OUT)],
            out_hbm.at[b, pl.ds(cc * BLK_OUT, BLK_OUT)],
        )
        return _

    lax.fori_loop(0, NUNITS, unit_body, None)


@jax.jit
def _hexcrop_sc(input_tensor, scal, mtab):
    mesh = plsc.VectorSubcoreMesh(
        core_axis_name="c", subcore_axis_name="s", num_cores=NC, num_subcores=NS
    )
    f = pl.kernel(
        _sc_body,
        out_type=jax.ShapeDtypeStruct((BATCH, CHANNELS * CH_OUT), jnp.float32),
        mesh=mesh,
        scratch_types=[
            pltpu.VMEM((2 * CH_CHUNK, CS, BUF_W), jnp.float32),
            pltpu.VMEM((OBUF_W,), jnp.float32),
            pltpu.VMEM((BPW, 16), jnp.int32),
            pltpu.VMEM((NT, 16), jnp.int32),
            pltpu.VMEM((NT, 16), jnp.float32),
            pltpu.VMEM((BPW * NT, 16), jnp.int32),
            pltpu.VMEM((BPW * NT, 16), jnp.float32),
            pltpu.SemaphoreType.DMA((2,)),
        ],
        compiler_params=pltpu.CompilerParams(
            use_tc_tiling_on_sc=False, needs_layout_passes=False
        ),
    )
    return f(input_tensor, scal, _COMBO, mtab)


def kernel(input_tensor, center_positions, mask, crop_mask):
    u = center_positions[:, 0].astype(jnp.int32)
    v = center_positions[:, 1].astype(jnp.int32)
    rs2 = jnp.clip(u - CC, 0, ENV - CS)
    du = (u - CC) - rs2
    cs2 = jnp.clip(v - CC, 0, ENV - FETCH_W) & ~15
    dv = (v - CC) - cs2
    scal = jnp.zeros((BATCH, 16), jnp.int32)
    scal = (
        scal.at[:, 0].set(rs2).at[:, 1].set(du).at[:, 2].set(cs2).at[:, 3].set(dv)
    )

    m_eff = jnp.where(mask, crop_mask, jnp.ones_like(crop_mask))
    mtab = m_eff[jnp.asarray(_I), jnp.asarray(_J)].reshape(NT, 16)

    out = _hexcrop_sc(input_tensor, scal, mtab)
    out = out.reshape(BATCH, CHANNELS, CS, CS)
    return (out, crop_mask)


# speedup vs baseline: 4.2297x; 1.0377x over previous
import numpy as np

import jax
import jax.numpy as jnp
from jax import lax
from jax.experimental import pallas as pl
from jax.experimental.pallas import tpu as pltpu
from jax.experimental.pallas import tpu_sc as plsc

BATCH = 128
CHANNELS = 32
ENV = 128
CS = 33
CC = 16
NC = 2
NS = 16
NW = NC * NS
BPW = BATCH // NW
CH_CHUNK = 8
N_CHUNKS = CHANNELS // CH_CHUNK
NUNITS = BPW * N_CHUNKS
FETCH_W = 48
BUF_W = 96
CH_OUT = CS * CS
BLK_OUT = CH_CHUNK * CH_OUT
NT = (CH_OUT + 15) // 16
OBUF_HALF = BLK_OUT + 24
OBUF_W = 2 * OBUF_HALF

_P = np.arange(NT * 16)
_Q = _P % CH_OUT
_I = (_Q // CS).astype(np.int32)
_J = (_Q % CS).astype(np.int32)
_COMBO = jnp.asarray((_I | (_J << 8)).reshape(NT, 16))


def _sc_body(
    x_hbm, scal_hbm, combo_hbm, mtab_hbm, out_hbm,
    buf, obuf, scal_v, combo_v, mtab_v, jctab, vmtab, sem, osem,
):
    wid = lax.axis_index("s") * NC + lax.axis_index("c")

    pltpu.sync_copy(combo_hbm, combo_v)
    pltpu.sync_copy(mtab_hbm, mtab_v)
    pltpu.sync_copy(scal_hbm.at[pl.ds(wid * BPW, BPW)], scal_v)

    lane = lax.iota(jnp.int32, 16)
    chd = jnp.where(lane > 0, 1, 0)

    def zero_row(i, _):
        for ch in range(2 * CH_CHUNK):
            for k in (0, 4, 5):
                buf[ch, i, pl.ds(16 * k, 16)] = jnp.zeros((16,), jnp.float32)
        return _

    lax.fori_loop(0, CS, zero_row, None)

    def tbl_batch(bi, _):
        srow = scal_v[bi]
        rs2 = srow[0]
        du = srow[1]
        dv = srow[3]
        rsdu = rs2 + du

        def table_body(t, _):
            c = combo_v[t]
            i_vec = c & 255
            j_vec = c >> 8
            jb = jnp.clip(i_vec + du, 0, CS - 1)
            col = j_vec + (CC + dv)
            jctab[bi * NT + t] = jb | (col << 8)
            img = i_vec + rsdu
            valid = jnp.logical_and(img >= 0, img < ENV)
            vmtab[bi * NT + t] = jnp.where(valid, mtab_v[t], 0.0)
            return _

        lax.fori_loop(0, NT, table_body, None)
        return _

    lax.fori_loop(0, BPW, tbl_batch, None)

    def band_copy(u, par):
        bi = u >> 2
        cc = u & (N_CHUNKS - 1)
        srow = scal_v[bi]
        rs2 = srow[0]
        cs2 = pl.multiple_of(srow[2], 16)
        b = wid * BPW + bi
        return pltpu.make_async_copy(
            x_hbm.at[
                b,
                pl.ds(cc * CH_CHUNK, CH_CHUNK),
                pl.ds(rs2, CS),
                pl.ds(cs2, FETCH_W),
            ],
            buf.at[pl.ds(par * CH_CHUNK, CH_CHUNK), :, pl.ds(CC, FETCH_W)],
            sem.at[par],
        )

    def out_copy(u, par):
        bi = u >> 2
        cc = u & (N_CHUNKS - 1)
        b = wid * BPW + bi
        return pltpu.make_async_copy(
            obuf.at[pl.ds(par * OBUF_HALF, BLK_OUT)],
            out_hbm.at[b, pl.ds(cc * BLK_OUT, BLK_OUT)],
            osem.at[par],
        )

    band_copy(0, 0).start()

    def unit_body(u, _):
        par = u & 1
        bi = u >> 2
        cc = u & (N_CHUNKS - 1)
        tb = bi * NT
        ob = par * OBUF_HALF

        @pl.when(u + 1 < NUNITS)
        def _start_next():
            band_copy(u + 1, 1 - par).start()

        band_copy(u, par).wait()

        @pl.when(u >= 2)
        def _drain_prev():
            out_copy(u - 2, par).wait()

        @plsc.parallel_loop(0, (NT - 1) * CH_CHUNK, unroll=8)
        def chunk_loop(m):
            ch = m & (CH_CHUNK - 1)
            t = m >> 3
            pk = jctab[tb + t]
            jb = pk & 255
            col = pk >> 8
            chsplat = jnp.full((16,), 0, jnp.int32) + (par * CH_CHUNK + ch)
            g = plsc.load_gather(buf, [chsplat, jb, col])
            obuf[pl.ds(ob + ch * CH_OUT + 16 * t, 16)] = g * vmtab[tb + t]

        @plsc.parallel_loop(0, CH_CHUNK, unroll=2)
        def tail_loop(ch):
            pk = jctab[tb + NT - 1]
            jb = pk & 255
            col = pk >> 8
            chsplat = jnp.full((16,), 0, jnp.int32) + (par * CH_CHUNK + ch)
            chv = jnp.minimum(chsplat + chd, par * CH_CHUNK + CH_CHUNK - 1)
            g = plsc.load_gather(buf, [chv, jb, col])
            obuf[pl.ds(ob + ch * CH_OUT + 16 * (NT - 1), 16)] = (
                g * vmtab[tb + NT - 1]
            )

        out_copy(u, par).start()
        return _

    lax.fori_loop(0, NUNITS, unit_body, None)
    out_copy(NUNITS - 2, 0).wait()
    out_copy(NUNITS - 1, 1).wait()


@jax.jit
def _hexcrop_sc(input_tensor, scal, mtab):
    mesh = plsc.VectorSubcoreMesh(
        core_axis_name="c", subcore_axis_name="s", num_cores=NC, num_subcores=NS
    )
    f = pl.kernel(
        _sc_body,
        out_type=jax.ShapeDtypeStruct((BATCH, CHANNELS * CH_OUT), jnp.float32),
        mesh=mesh,
        scratch_types=[
            pltpu.VMEM((2 * CH_CHUNK, CS, BUF_W), jnp.float32),
            pltpu.VMEM((OBUF_W,), jnp.float32),
            pltpu.VMEM((BPW, 16), jnp.int32),
            pltpu.VMEM((NT, 16), jnp.int32),
            pltpu.VMEM((NT, 16), jnp.float32),
            pltpu.VMEM((BPW * NT, 16), jnp.int32),
            pltpu.VMEM((BPW * NT, 16), jnp.float32),
            pltpu.SemaphoreType.DMA((2,)),
            pltpu.SemaphoreType.DMA((2,)),
        ],
        compiler_params=pltpu.CompilerParams(
            use_tc_tiling_on_sc=False, needs_layout_passes=False
        ),
    )
    return f(input_tensor, scal, _COMBO, mtab)


def kernel(input_tensor, center_positions, mask, crop_mask):
    u = center_positions[:, 0].astype(jnp.int32)
    v = center_positions[:, 1].astype(jnp.int32)
    rs2 = jnp.clip(u - CC, 0, ENV - CS)
    du = (u - CC) - rs2
    cs2 = jnp.clip(v - CC, 0, ENV - FETCH_W) & ~15
    dv = (v - CC) - cs2
    scal = jnp.zeros((BATCH, 16), jnp.int32)
    scal = (
        scal.at[:, 0].set(rs2).at[:, 1].set(du).at[:, 2].set(cs2).at[:, 3].set(dv)
    )

    m_eff = jnp.where(mask, crop_mask, jnp.ones_like(crop_mask))
    mtab = m_eff[jnp.asarray(_I), jnp.asarray(_J)].reshape(NT, 16)

    out = _hexcrop_sc(input_tensor, scal, mtab)
    out = out.reshape(BATCH, CHANNELS, CS, CS)
    return (out, crop_mask)


# speedup vs baseline: 4.2365x; 1.0016x over previous
import numpy as np

import jax
import jax.numpy as jnp
from jax import lax
from jax.experimental import pallas as pl
from jax.experimental.pallas import tpu as pltpu
from jax.experimental.pallas import tpu_sc as plsc

BATCH = 128
CHANNELS = 32
ENV = 128
CS = 33
CC = 16
NC = 2
NS = 16
NW = NC * NS
BPW = BATCH // NW
CH_CHUNK = 8
N_CHUNKS = CHANNELS // CH_CHUNK
NUNITS = BPW * N_CHUNKS
FETCH_W = 48
BUF_W = 96
CH_OUT = CS * CS
BLK_OUT = CH_CHUNK * CH_OUT
NT = (CH_OUT + 15) // 16
OBUF_HALF = BLK_OUT + 24
OBUF_W = 2 * OBUF_HALF

_P = np.arange(NT * 16)
_Q = _P % CH_OUT
_I = (_Q // CS).astype(np.int32)
_J = (_Q % CS).astype(np.int32)
_COMBO = jnp.asarray((_I | (_J << 8)).reshape(NT, 16))


def _sc_body(
    x_hbm, scal_hbm, combo_hbm, mtab_hbm, out_hbm,
    buf, obuf, scal_v, combo_v, mtab_v, jctab, vmtab, sem, osem,
):
    wid = lax.axis_index("s") * NC + lax.axis_index("c")

    pltpu.sync_copy(combo_hbm, combo_v)
    pltpu.sync_copy(mtab_hbm, mtab_v)
    pltpu.sync_copy(scal_hbm.at[pl.ds(wid * BPW, BPW)], scal_v)

    def band_copy(u, par):
        bi = u >> 2
        cc = u & (N_CHUNKS - 1)
        srow = scal_v[bi]
        rs2 = srow[0]
        cs2 = pl.multiple_of(srow[2], 16)
        b = wid * BPW + bi
        return pltpu.make_async_copy(
            x_hbm.at[
                b,
                pl.ds(cc * CH_CHUNK, CH_CHUNK),
                pl.ds(rs2, CS),
                pl.ds(cs2, FETCH_W),
            ],
            buf.at[pl.ds(par * CH_CHUNK, CH_CHUNK), :, pl.ds(CC, FETCH_W)],
            sem.at[par],
        )

    band_copy(0, 0).start()

    lane = lax.iota(jnp.int32, 16)
    chd = jnp.where(lane > 0, 1, 0)

    def zero_row(i, _):
        for ch in range(2 * CH_CHUNK):
            for k in (0, 4, 5):
                buf[ch, i, pl.ds(16 * k, 16)] = jnp.zeros((16,), jnp.float32)
        return _

    lax.fori_loop(0, CS, zero_row, None)

    def tbl_batch(bi, _):
        srow = scal_v[bi]
        rs2 = srow[0]
        du = srow[1]
        dv = srow[3]
        rsdu = rs2 + du

        def table_body(t, _):
            c = combo_v[t]
            i_vec = c & 255
            j_vec = c >> 8
            jb = jnp.clip(i_vec + du, 0, CS - 1)
            col = j_vec + (CC + dv)
            jctab[bi * NT + t] = jb | (col << 8)
            img = i_vec + rsdu
            valid = jnp.logical_and(img >= 0, img < ENV)
            vmtab[bi * NT + t] = jnp.where(valid, mtab_v[t], 0.0)
            return _

        lax.fori_loop(0, NT, table_body, None)
        return _

    lax.fori_loop(0, BPW, tbl_batch, None)

    def out_copy(u, par):
        bi = u >> 2
        cc = u & (N_CHUNKS - 1)
        b = wid * BPW + bi
        return pltpu.make_async_copy(
            obuf.at[pl.ds(par * OBUF_HALF, BLK_OUT)],
            out_hbm.at[b, pl.ds(cc * BLK_OUT, BLK_OUT)],
            osem.at[par],
        )

    def unit_body(u, _):
        par = u & 1
        bi = u >> 2
        cc = u & (N_CHUNKS - 1)
        tb = bi * NT
        ob = par * OBUF_HALF

        @pl.when(u + 1 < NUNITS)
        def _start_next():
            band_copy(u + 1, 1 - par).start()

        band_copy(u, par).wait()

        @pl.when(u >= 2)
        def _drain_prev():
            out_copy(u - 2, par).wait()

        @plsc.parallel_loop(0, (NT - 1) * CH_CHUNK, unroll=16)
        def chunk_loop(m):
            ch = m & (CH_CHUNK - 1)
            t = m >> 3
            pk = jctab[tb + t]
            jb = pk & 255
            col = pk >> 8
            chsplat = jnp.full((16,), 0, jnp.int32) + (par * CH_CHUNK + ch)
            g = plsc.load_gather(buf, [chsplat, jb, col])
            obuf[pl.ds(ob + ch * CH_OUT + 16 * t, 16)] = g * vmtab[tb + t]

        @plsc.parallel_loop(0, CH_CHUNK, unroll=2)
        def tail_loop(ch):
            pk = jctab[tb + NT - 1]
            jb = pk & 255
            col = pk >> 8
            chsplat = jnp.full((16,), 0, jnp.int32) + (par * CH_CHUNK + ch)
            chv = jnp.minimum(chsplat + chd, par * CH_CHUNK + CH_CHUNK - 1)
            g = plsc.load_gather(buf, [chv, jb, col])
            obuf[pl.ds(ob + ch * CH_OUT + 16 * (NT - 1), 16)] = (
                g * vmtab[tb + NT - 1]
            )

        out_copy(u, par).start()
        return _

    lax.fori_loop(0, NUNITS, unit_body, None)
    out_copy(NUNITS - 2, 0).wait()
    out_copy(NUNITS - 1, 1).wait()


@jax.jit
def _hexcrop_sc(input_tensor, scal, mtab):
    mesh = plsc.VectorSubcoreMesh(
        core_axis_name="c", subcore_axis_name="s", num_cores=NC, num_subcores=NS
    )
    f = pl.kernel(
        _sc_body,
        out_type=jax.ShapeDtypeStruct((BATCH, CHANNELS * CH_OUT), jnp.float32),
        mesh=mesh,
        scratch_types=[
            pltpu.VMEM((2 * CH_CHUNK, CS, BUF_W), jnp.float32),
            pltpu.VMEM((OBUF_W,), jnp.float32),
            pltpu.VMEM((BPW, 16), jnp.int32),
            pltpu.VMEM((NT, 16), jnp.int32),
            pltpu.VMEM((NT, 16), jnp.float32),
            pltpu.VMEM((BPW * NT, 16), jnp.int32),
            pltpu.VMEM((BPW * NT, 16), jnp.float32),
            pltpu.SemaphoreType.DMA((2,)),
            pltpu.SemaphoreType.DMA((2,)),
        ],
        compiler_params=pltpu.CompilerParams(
            use_tc_tiling_on_sc=False, needs_layout_passes=False
        ),
    )
    return f(input_tensor, scal, _COMBO, mtab)


def kernel(input_tensor, center_positions, mask, crop_mask):
    u = center_positions[:, 0].astype(jnp.int32)
    v = center_positions[:, 1].astype(jnp.int32)
    rs2 = jnp.clip(u - CC, 0, ENV - CS)
    du = (u - CC) - rs2
    cs2 = jnp.clip(v - CC, 0, ENV - FETCH_W) & ~15
    dv = (v - CC) - cs2
    scal = jnp.zeros((BATCH, 16), jnp.int32)
    scal = (
        scal.at[:, 0].set(rs2).at[:, 1].set(du).at[:, 2].set(cs2).at[:, 3].set(dv)
    )

    m_eff = jnp.where(mask, crop_mask, jnp.ones_like(crop_mask))
    mtab = m_eff[jnp.asarray(_I), jnp.asarray(_J)].reshape(NT, 16)

    out = _hexcrop_sc(input_tensor, scal, mtab)
    out = out.reshape(BATCH, CHANNELS, CS, CS)
    return (out, crop_mask)
